# baseline (speedup 1.0000x reference)
import functools
import math

import jax
import jax.numpy as jnp
from jax import lax
from jax.experimental import pallas as pl
from jax.experimental.pallas import tpu as pltpu
from jax.experimental.pallas import tpu_sc as plsc

D_MODEL = 128
SCALE = math.sqrt(128.0)
NUM_CORES = 2
NUM_SUBCORES = 16
NUM_WORKERS = NUM_CORES * NUM_SUBCORES
SEQ = 50


@functools.partial(jax.jit, static_argnames=("batches",))
def _embed_sc(idx, table, batches):
    b_per_w = batches // NUM_WORKERS
    npairs = b_per_w // 2

    @functools.partial(
        pl.kernel,
        out_type=jax.ShapeDtypeStruct((batches, SEQ, D_MODEL), jnp.float32),
        mesh=plsc.VectorSubcoreMesh(core_axis_name="c", subcore_axis_name="s"),
        compiler_params=pltpu.CompilerParams(use_tc_tiling_on_sc=True),
        scratch_types=[
            pltpu.VMEM((b_per_w, SEQ), jnp.int32),
            pltpu.VMEM((SEQ, D_MODEL), jnp.float32),
            pltpu.VMEM((SEQ, D_MODEL), jnp.float32),
            pltpu.SemaphoreType.DMA,
            pltpu.SemaphoreType.DMA,
            pltpu.SemaphoreType.DMA,
            pltpu.SemaphoreType.DMA,
        ],
    )
    def k(idx_hbm, table_hbm, out_hbm, idx_v, buf0, buf1, g0sem, g1sem,
          o0sem, o1sem):
        wid = lax.axis_index("s") * NUM_CORES + lax.axis_index("c")
        pltpu.sync_copy(idx_hbm.at[wid], idx_v)
        base = wid * b_per_w

        def scale(buf):
            def body(q, c2):
                r0 = q * 5
                for r in range(5):
                    for j in range(D_MODEL // 16):
                        sl = pl.ds(j * 16, 16)
                        buf[r0 + r, sl] = buf[r0 + r, sl] * SCALE
                return c2

            lax.fori_loop(0, SEQ // 5, body, 0)

        def gather_start(g, buf, sem):
            pltpu.async_copy(table_hbm.at[idx_v.at[g]], buf, sem)

        def gather_wait(g, buf, sem):
            pltpu.make_async_copy(table_hbm.at[idx_v.at[g]], buf, sem).wait()

        def put_start(g, buf, sem):
            pltpu.async_copy(buf, out_hbm.at[base + g], sem)

        def put_wait(g, buf, sem):
            pltpu.make_async_copy(buf, out_hbm.at[base + g], sem).wait()

        gather_start(0, buf0, g0sem)
        gather_start(1, buf1, g1sem)

        def pair(h, carry):
            g0 = h * 2
            g1 = g0 + 1
            gather_wait(g0, buf0, g0sem)
            scale(buf0)
            put_start(g0, buf0, o0sem)
            gather_wait(g1, buf1, g1sem)
            scale(buf1)
            put_start(g1, buf1, o1sem)

            @pl.when(h + 1 < npairs)
            def _():
                put_wait(g0, buf0, o0sem)
                gather_start(g0 + 2, buf0, g0sem)
                put_wait(g1, buf1, o1sem)
                gather_start(g1 + 2, buf1, g1sem)

            return carry

        lax.fori_loop(0, npairs, pair, 0)
        put_wait(b_per_w - 2, buf0, o0sem)
        put_wait(b_per_w - 1, buf1, o1sem)

    return k(idx, table)


NCHUNK = 4


def kernel(x, word_emb):
    batches = x.shape[0]
    bc = batches // NCHUNK
    idx = x.reshape(NCHUNK, NUM_WORKERS, bc // NUM_WORKERS, SEQ).astype(
        jnp.int32)
    outs = [_embed_sc(idx[c], word_emb, bc) for c in range(NCHUNK)]
    return jnp.concatenate(outs, axis=0)


# speedup vs baseline: 2.0644x; 2.0644x over previous
import functools
import math

import jax
import jax.numpy as jnp
from jax import lax
from jax.experimental import pallas as pl
from jax.experimental.pallas import tpu as pltpu
from jax.experimental.pallas import tpu_sc as plsc

D_MODEL = 128
SCALE = math.sqrt(128.0)
NUM_CORES = 2
NUM_SUBCORES = 16
NUM_WORKERS = NUM_CORES * NUM_SUBCORES
SEQ = 50
BPS = 2
ROWS = BPS * SEQ
IPAD = 104


@functools.partial(jax.jit, static_argnames=("batches",))
def _embed_sc(idx, table, batches):
    b_per_w = batches // NUM_WORKERS
    nsteps = b_per_w // BPS

    @functools.partial(
        pl.kernel,
        out_type=jax.ShapeDtypeStruct((batches, SEQ, D_MODEL), jnp.float32),
        mesh=plsc.VectorSubcoreMesh(core_axis_name="c", subcore_axis_name="s"),
        scratch_types=[
            pltpu.VMEM((nsteps, IPAD), jnp.int32),
            pltpu.VMEM((4, ROWS, D_MODEL), jnp.float32),
            [pltpu.SemaphoreType.DMA] * 4,
            [pltpu.SemaphoreType.DMA] * 4,
        ],
    )
    def k(idx_hbm, table_hbm, out_hbm, idx_v, bufs, gsems, osems):
        wid = lax.axis_index("s") * NUM_CORES + lax.axis_index("c")
        pltpu.sync_copy(idx_hbm.at[wid], idx_v)
        base = wid * b_per_w

        def scale(buf):
            def body(q, c2):
                r0 = q * 5
                for r in range(5):
                    for j in range(D_MODEL // 16):
                        sl = pl.ds(j * 16, 16)
                        buf[r0 + r, sl] = buf[r0 + r, sl] * SCALE
                return c2

            lax.fori_loop(0, ROWS // 5, body, 0)

        def gather_start(g, buf, sem):
            pltpu.async_copy(
                table_hbm.at[idx_v.at[g, pl.ds(0, ROWS)]], buf, sem)

        def gather_wait(g, buf, sem):
            pltpu.make_async_copy(
                table_hbm.at[idx_v.at[g, pl.ds(0, ROWS)]], buf, sem).wait()

        def put_start(g, buf, sem):
            b0 = base + g * BPS
            pltpu.async_copy(buf.at[pl.ds(0, SEQ)], out_hbm.at[b0], sem)
            pltpu.async_copy(buf.at[pl.ds(SEQ, SEQ)], out_hbm.at[b0 + 1], sem)

        def put_wait(g, buf, sem):
            b0 = base + g * BPS
            pltpu.make_async_copy(
                buf.at[pl.ds(0, SEQ)], out_hbm.at[b0], sem).wait()
            pltpu.make_async_copy(
                buf.at[pl.ds(SEQ, SEQ)], out_hbm.at[b0 + 1], sem).wait()

        gather_start(0, bufs.at[0], gsems[0])
        gather_start(1, bufs.at[1], gsems[1])

        def quad(q, carry):
            g0 = q * 4
            for i in range(4):
                g = g0 + i
                buf = bufs.at[i]
                gather_wait(g, buf, gsems[i])

                @pl.when(g >= 2)
                def _():
                    put_wait(g - 2, bufs.at[(i + 2) % 4], osems[(i + 2) % 4])

                @pl.when(g + 2 < nsteps)
                def _():
                    gather_start(
                        g + 2, bufs.at[(i + 2) % 4], gsems[(i + 2) % 4])

                scale(buf)
                put_start(g, buf, osems[i])
            return carry

        lax.fori_loop(0, nsteps // 4, quad, 0)
        put_wait(nsteps - 2, bufs.at[2], osems[2])
        put_wait(nsteps - 1, bufs.at[3], osems[3])

    return k(idx, table)


def kernel(x, word_emb):
    batches = x.shape[0]
    b_per_w = batches // NUM_WORKERS
    nsteps = b_per_w // BPS
    xr = x.reshape(NUM_WORKERS, nsteps, ROWS).astype(jnp.int32)
    idx = jnp.pad(xr, ((0, 0), (0, 0), (0, IPAD - ROWS)))
    return _embed_sc(idx, word_emb, batches)
